# baseline (speedup 1.0000x reference)
import functools

import jax
import jax.numpy as jnp
from jax import lax
from jax.experimental import pallas as pl
from jax.experimental.pallas import tpu as pltpu
from jax.experimental.pallas import tpu_sc as plsc

N = 10000
NP = 10240
E = 320000
EP = 327680
EROWS = EP // 128
CHUNKS2 = EROWS // 32
CHUNKS1 = EROWS // 16
STRIPE = NP // 16

_mesh = plsc.VectorSubcoreMesh(core_axis_name="c", subcore_axis_name="s")


@functools.partial(
    pl.kernel,
    out_type=jax.ShapeDtypeStruct((2, NP), jnp.float32),
    mesh=_mesh,
    scratch_types=[
        pltpu.VMEM((CHUNKS2, 128), jnp.int32),
        pltpu.VMEM((128,), jnp.float32),
        pltpu.VMEM((STRIPE,), jnp.float32),
        pltpu.VMEM_SHARED((NP,), jnp.float32),
    ],
)
def _sc_degree(dst_hbm, out_hbm, dst_v, ones_v, st_v, acc_sh):
    c = lax.axis_index("c")
    s = lax.axis_index("s")
    tile = c * 16 + s
    pltpu.sync_copy(dst_hbm.at[pl.ds(tile * CHUNKS2, CHUNKS2)], dst_v)
    one16 = jnp.full((16,), 1.0, jnp.float32)
    zero16 = jnp.zeros((16,), jnp.float32)
    for k in range(8):
        ones_v[pl.ds(k * 16, 16)] = one16

    def _zero(i, carry):
        st_v[pl.ds(i * 16, 16)] = zero16
        return carry

    lax.fori_loop(0, STRIPE // 16, _zero, 0)
    pltpu.sync_copy(st_v, acc_sh.at[pl.ds(s * STRIPE, STRIPE)])
    plsc.subcore_barrier()

    def _scatter(ch, carry):
        pltpu.sync_copy(ones_v, acc_sh.at[dst_v.at[ch]], add=True)
        return carry

    lax.fori_loop(0, CHUNKS2, _scatter, 0)
    plsc.subcore_barrier()
    pltpu.sync_copy(acc_sh.at[pl.ds(s * STRIPE, STRIPE)], st_v)
    pltpu.sync_copy(st_v, out_hbm.at[c, pl.ds(s * STRIPE, STRIPE)])


def _unpack_idx(wv, row, colbase, out_v):
    for j in range(4):
        w16 = wv[row, pl.ds(colbase + j * 16, 16)]
        out_v[pl.ds(j * 16, 16)] = w16 & 0xFFFF
        out_v[pl.ds(64 + j * 16, 16)] = lax.shift_right_logical(w16, 16)


@functools.partial(
    pl.kernel,
    out_type=jax.ShapeDtypeStruct((2, NP, 128), jnp.float32),
    mesh=_mesh,
    scratch_types=[
        pltpu.VMEM((CHUNKS2 // 2, 128), jnp.int32),
        pltpu.VMEM((CHUNKS2 // 2, 128), jnp.int32),
        pltpu.VMEM((128,), jnp.int32),
        pltpu.VMEM((128,), jnp.int32),
        pltpu.VMEM((128,), jnp.int32),
        pltpu.VMEM((128,), jnp.int32),
        pltpu.VMEM((128, 128), jnp.float32),
        pltpu.VMEM((128, 128), jnp.float32),
        pltpu.VMEM_SHARED((NP, 128), jnp.float32),
        pltpu.SemaphoreType.DMA,
        pltpu.SemaphoreType.DMA,
        pltpu.SemaphoreType.DMA,
        pltpu.SemaphoreType.DMA,
    ],
)
def _sc_row_agg(srcw_hbm, dstw_hbm, y_hbm, out_hbm,
                srcw_v, dstw_v, sidx0, sidx1, didx0, didx1,
                buf0, buf1, acc_sh, gsem0, gsem1, ssem0, ssem1):
    c = lax.axis_index("c")
    s = lax.axis_index("s")
    tile = c * 16 + s
    nrows = CHUNKS2 // 2
    pltpu.sync_copy(srcw_hbm.at[pl.ds(tile * nrows, nrows)], srcw_v)
    pltpu.sync_copy(dstw_hbm.at[pl.ds(tile * nrows, nrows)], dstw_v)

    zero16 = jnp.zeros((16,), jnp.float32)

    def _zero(i, carry):
        for k in range(8):
            buf0[i, pl.ds(k * 16, 16)] = zero16
        return carry

    lax.fori_loop(0, 128, _zero, 0)
    for k in range(STRIPE // 128):
        pltpu.sync_copy(buf0, acc_sh.at[pl.ds(s * STRIPE + k * 128, 128)])
    plsc.subcore_barrier()

    bufs = (buf0, buf1)
    gsem = (gsem0, gsem1)
    ssem = (ssem0, ssem1)
    sidx = (sidx0, sidx1)
    didx = (didx0, didx1)

    def _gather(ch, row, colbase, b):
        _unpack_idx(srcw_v, row, colbase, sidx[b])
        pltpu.async_copy(y_hbm.at[sidx[b]], bufs[b], gsem[b])

    def _wait_gather(b):
        pltpu.make_async_copy(y_hbm.at[sidx[b]], bufs[b], gsem[b]).wait()

    def _scatter(ch, row, colbase, b):
        _unpack_idx(dstw_v, row, colbase, didx[b])
        pltpu.async_copy(bufs[b], acc_sh.at[didx[b]], ssem[b], add=True)

    def _wait_scatter(b):
        pltpu.make_async_copy(bufs[b], acc_sh.at[didx[b]], ssem[b]).wait()

    _gather(0, 0, 0, 0)
    _wait_gather(0)
    _scatter(0, 0, 0, 0)
    _gather(1, 0, 64, 1)

    def _body(g, carry):
        _wait_gather(1)
        _unpack_idx(dstw_v, g, 64, didx[1])
        pltpu.async_copy(bufs[1], acc_sh.at[didx[1]], ssem[1], add=True)
        _wait_scatter(0)
        _unpack_idx(srcw_v, g + 1, 0, sidx[0])
        pltpu.async_copy(y_hbm.at[sidx[0]], bufs[0], gsem[0])

        _wait_gather(0)
        _unpack_idx(dstw_v, g + 1, 0, didx[0])
        pltpu.async_copy(bufs[0], acc_sh.at[didx[0]], ssem[0], add=True)
        _wait_scatter(1)
        _unpack_idx(srcw_v, g + 1, 64, sidx[1])
        pltpu.async_copy(y_hbm.at[sidx[1]], bufs[1], gsem[1])
        return carry

    lax.fori_loop(0, CHUNKS2 // 2 - 1, _body, 0)
    _wait_gather(1)
    _scatter(CHUNKS2 - 1, CHUNKS2 // 2 - 1, 64, 1)
    _wait_scatter(0)
    _wait_scatter(1)
    plsc.subcore_barrier()
    for k in range(STRIPE // 128):
        pltpu.sync_copy(acc_sh.at[pl.ds(s * STRIPE + k * 128, 128)], buf0)
        pltpu.sync_copy(buf0, out_hbm.at[c, pl.ds(s * STRIPE + k * 128, 128)])


@functools.partial(
    pl.kernel,
    out_type=jax.ShapeDtypeStruct((NP,), jnp.float32),
    mesh=_mesh,
    scratch_types=[
        pltpu.VMEM((CHUNKS1, 128), jnp.int32),
        pltpu.VMEM((CHUNKS1, 128), jnp.int32),
        pltpu.VMEM((128,), jnp.float32),
        pltpu.VMEM((128,), jnp.float32),
        pltpu.VMEM((STRIPE,), jnp.float32),
        pltpu.VMEM((STRIPE,), jnp.float32),
        pltpu.VMEM((STRIPE,), jnp.float32),
        pltpu.VMEM((STRIPE,), jnp.float32),
        pltpu.VMEM((16,), jnp.float32),
        pltpu.VMEM_SHARED((NP,), jnp.float32),
        pltpu.VMEM_SHARED((NP,), jnp.float32),
        pltpu.SemaphoreType.DMA,
        pltpu.SemaphoreType.DMA,
    ],
)
def _sc_scalar_agg(src_hbm, dst_hbm, t_hbm, dis_hbm, b2_hbm, out_hbm,
                   src_v, dst_v, vals0, vals1, av, dv, tv, ov, b2_v,
                   t_sh, acc_sh, sem0, sem1):
    c = lax.axis_index("c")
    s = lax.axis_index("s")

    @pl.when(c == 0)
    def _():
        pltpu.sync_copy(src_hbm.at[pl.ds(s * CHUNKS1, CHUNKS1)], src_v)
        pltpu.sync_copy(dst_hbm.at[pl.ds(s * CHUNKS1, CHUNKS1)], dst_v)
        pltpu.sync_copy(b2_hbm, b2_v)
        pltpu.sync_copy(t_hbm.at[pl.ds(s * STRIPE, STRIPE)], tv)
        pltpu.sync_copy(tv, t_sh.at[pl.ds(s * STRIPE, STRIPE)])
        zero16 = jnp.zeros((16,), jnp.float32)

        def _zero(i, carry):
            av[pl.ds(i * 16, 16)] = zero16
            return carry

        lax.fori_loop(0, STRIPE // 16, _zero, 0)
        pltpu.sync_copy(av, acc_sh.at[pl.ds(s * STRIPE, STRIPE)])
        plsc.subcore_barrier()

        vals = (vals0, vals1)
        sems = (sem0, sem1)
        for b in range(2):
            pltpu.async_copy(t_sh.at[src_v.at[b]], vals[b], sems[b])

        def _scatter(g, carry):
            for b in range(2):
                ch = g * 2 + b
                pltpu.make_async_copy(
                    t_sh.at[src_v.at[ch]], vals[b], sems[b]).wait()
                pltpu.sync_copy(vals[b], acc_sh.at[dst_v.at[ch]], add=True)

                @pl.when(ch + 2 < CHUNKS1)
                def _():
                    pltpu.async_copy(t_sh.at[src_v.at[ch + 2]], vals[b], sems[b])
            return carry

        lax.fori_loop(0, CHUNKS1 // 2, _scatter, 0)
        plsc.subcore_barrier()
        pltpu.sync_copy(acc_sh.at[pl.ds(s * STRIPE, STRIPE)], av)
        pltpu.sync_copy(dis_hbm.at[pl.ds(s * STRIPE, STRIPE)], dv)
        b2r = b2_v[...]

        def _combine(i, carry):
            a16 = av[pl.ds(i * 16, 16)]
            d16 = dv[pl.ds(i * 16, 16)]
            t16 = tv[pl.ds(i * 16, 16)]
            ov[pl.ds(i * 16, 16)] = d16 * (a16 + t16) + b2r
            return carry

        lax.fori_loop(0, STRIPE // 16, _combine, 0)
        pltpu.sync_copy(ov, out_hbm.at[pl.ds(s * STRIPE, STRIPE)])


def _tc1_body(x_ref, w_ref, dg_ref, y_ref, dis_ref):
    deg = 1.0 + dg_ref[:, 0:1] + dg_ref[:, 1:2]
    dis = lax.rsqrt(deg)
    xw = jnp.dot(x_ref[...], w_ref[...], preferred_element_type=jnp.float32)
    y_ref[...] = xw * dis
    dis_ref[...] = dis


def _tc1(xp, W1, degp_t):
    blk = 1024
    grid = NP // blk
    return pl.pallas_call(
        _tc1_body,
        grid=(grid,),
        in_specs=[
            pl.BlockSpec((blk, 128), lambda i: (i, 0)),
            pl.BlockSpec((128, 128), lambda i: (0, 0)),
            pl.BlockSpec((blk, 2), lambda i: (i, 0)),
        ],
        out_specs=[
            pl.BlockSpec((blk, 128), lambda i: (i, 0)),
            pl.BlockSpec((blk, 1), lambda i: (i, 0)),
        ],
        out_shape=[
            jax.ShapeDtypeStruct((NP, 128), jnp.float32),
            jax.ShapeDtypeStruct((NP, 1), jnp.float32),
        ],
    )(xp, W1, degp_t)


def _tc2_body(ap_ref, y_ref, dis_ref, b1_ref, w2_ref, t_ref):
    agg = ap_ref[0] + ap_ref[1] + y_ref[...]
    h = jnp.maximum(agg * dis_ref[...] + b1_ref[...], 0.0)
    srow = jnp.sum(h * w2_ref[...], axis=1, keepdims=True)
    t_ref[...] = srow * dis_ref[...]


def _tc2(aggp, y, dis, b1r, w2r):
    blk = 1024
    grid = NP // blk
    return pl.pallas_call(
        _tc2_body,
        grid=(grid,),
        in_specs=[
            pl.BlockSpec((2, blk, 128), lambda i: (0, i, 0)),
            pl.BlockSpec((blk, 128), lambda i: (i, 0)),
            pl.BlockSpec((blk, 1), lambda i: (i, 0)),
            pl.BlockSpec((1, 128), lambda i: (0, 0)),
            pl.BlockSpec((1, 128), lambda i: (0, 0)),
        ],
        out_specs=pl.BlockSpec((blk, 1), lambda i: (i, 0)),
        out_shape=jax.ShapeDtypeStruct((NP, 1), jnp.float32),
    )(aggp, y, dis, b1r, w2r)


def _pack2x16(idx2d):
    w = idx2d[:, :64] | (idx2d[:, 64:] << 16)
    return w.reshape(EROWS // 2, 128)


@jax.jit
def kernel(x, edge_index, W1, b1, W2, b2):
    pad = (jnp.arange(EP - E, dtype=jnp.int32) % (NP - N)) + N
    srcp = jnp.concatenate([edge_index[0], pad]).reshape(EROWS, 128)
    dstp = jnp.concatenate([edge_index[1], pad]).reshape(EROWS, 128)
    srcw = _pack2x16(srcp)
    dstw = _pack2x16(dstp)
    xp = jnp.pad(x, ((0, NP - N), (0, 0)))

    degp = _sc_degree(dstp)
    y, dis = _tc1(xp, W1, jnp.transpose(degp))
    aggp = _sc_row_agg(srcw, dstw, y)
    t = _tc2(aggp, y, dis, b1.reshape(1, 128), W2.reshape(1, 128))
    out_full = _sc_scalar_agg(
        srcp, dstp, t.reshape(-1), dis.reshape(-1),
        jnp.broadcast_to(b2, (16,)))
    return out_full[:N]


# speedup vs baseline: 1.0958x; 1.0958x over previous
import functools

import jax
import jax.numpy as jnp
from jax import lax
from jax.experimental import pallas as pl
from jax.experimental.pallas import tpu as pltpu
from jax.experimental.pallas import tpu_sc as plsc

N = 10000
NP = 10240
E = 320000
EP = 327680
EROWS = EP // 128
CHUNKS2 = EROWS // 32
CHUNKS1 = EROWS // 16
STRIPE = NP // 16

_mesh = plsc.VectorSubcoreMesh(core_axis_name="c", subcore_axis_name="s")


@functools.partial(
    pl.kernel,
    out_type=jax.ShapeDtypeStruct((2, NP), jnp.float32),
    mesh=_mesh,
    scratch_types=[
        pltpu.VMEM((CHUNKS2, 128), jnp.int32),
        pltpu.VMEM((128,), jnp.float32),
        pltpu.VMEM((STRIPE,), jnp.float32),
        pltpu.VMEM_SHARED((NP,), jnp.float32),
    ],
)
def _sc_degree(dst_hbm, out_hbm, dst_v, ones_v, st_v, acc_sh):
    c = lax.axis_index("c")
    s = lax.axis_index("s")
    tile = c * 16 + s
    pltpu.sync_copy(dst_hbm.at[pl.ds(tile * CHUNKS2, CHUNKS2)], dst_v)
    one16 = jnp.full((16,), 1.0, jnp.float32)
    zero16 = jnp.zeros((16,), jnp.float32)
    for k in range(8):
        ones_v[pl.ds(k * 16, 16)] = one16

    def _zero(i, carry):
        st_v[pl.ds(i * 16, 16)] = zero16
        return carry

    lax.fori_loop(0, STRIPE // 16, _zero, 0)
    pltpu.sync_copy(st_v, acc_sh.at[pl.ds(s * STRIPE, STRIPE)])
    plsc.subcore_barrier()

    def _scatter(ch, carry):
        pltpu.sync_copy(ones_v, acc_sh.at[dst_v.at[ch]], add=True)
        return carry

    lax.fori_loop(0, CHUNKS2, _scatter, 0)
    plsc.subcore_barrier()
    pltpu.sync_copy(acc_sh.at[pl.ds(s * STRIPE, STRIPE)], st_v)
    pltpu.sync_copy(st_v, out_hbm.at[c, pl.ds(s * STRIPE, STRIPE)])


def _unpack_idx(wv, row, colbase, out_v):
    for j in range(4):
        w16 = wv[row, pl.ds(colbase + j * 16, 16)]
        out_v[pl.ds(j * 16, 16)] = w16 & 0xFFFF
        out_v[pl.ds(64 + j * 16, 16)] = lax.shift_right_logical(w16, 16)


@functools.partial(
    pl.kernel,
    out_type=jax.ShapeDtypeStruct((2, NP, 128), jnp.float32),
    mesh=_mesh,
    scratch_types=[
        pltpu.VMEM((CHUNKS2 // 2, 128), jnp.int32),
        pltpu.VMEM((CHUNKS2 // 2, 128), jnp.int32),
        pltpu.VMEM((128,), jnp.int32),
        pltpu.VMEM((128,), jnp.int32),
        pltpu.VMEM((128,), jnp.int32),
        pltpu.VMEM((128,), jnp.int32),
        pltpu.VMEM((128, 128), jnp.float32),
        pltpu.VMEM((128, 128), jnp.float32),
        pltpu.VMEM_SHARED((NP, 128), jnp.float32),
        pltpu.SemaphoreType.DMA,
        pltpu.SemaphoreType.DMA,
        pltpu.SemaphoreType.DMA,
        pltpu.SemaphoreType.DMA,
    ],
)
def _sc_row_agg(srcw_hbm, dstw_hbm, y_hbm, out_hbm,
                srcw_v, dstw_v, sidx0, sidx1, didx0, didx1,
                buf0, buf1, acc_sh, gsem0, gsem1, ssem0, ssem1):
    c = lax.axis_index("c")
    s = lax.axis_index("s")
    tile = c * 16 + s
    nrows = CHUNKS2 // 2
    pltpu.sync_copy(srcw_hbm.at[pl.ds(tile * nrows, nrows)], srcw_v)
    pltpu.sync_copy(dstw_hbm.at[pl.ds(tile * nrows, nrows)], dstw_v)

    zero16 = jnp.zeros((16,), jnp.float32)

    def _zero(i, carry):
        for k in range(8):
            buf0[i, pl.ds(k * 16, 16)] = zero16
        return carry

    lax.fori_loop(0, 128, _zero, 0)
    for k in range(STRIPE // 128):
        pltpu.sync_copy(buf0, acc_sh.at[pl.ds(s * STRIPE + k * 128, 128)])
    plsc.subcore_barrier()

    bufs = (buf0, buf1)
    gsem = (gsem0, gsem1)
    sidx = (sidx0, sidx1)
    didx = (didx0, didx1)
    for b in range(2):
        _unpack_idx(srcw_v, 0, b * 64, sidx[b])
        pltpu.async_copy(y_hbm.at[sidx[b]], bufs[b], gsem[b])

    def _body(g, carry):
        for b in range(2):
            ch = g * 2 + b
            pltpu.make_async_copy(y_hbm.at[sidx[b]], bufs[b], gsem[b]).wait()
            _unpack_idx(dstw_v, g, b * 64, didx[b])
            pltpu.sync_copy(bufs[b], acc_sh.at[didx[b]], add=True)

            @pl.when(ch + 2 < CHUNKS2)
            def _():
                _unpack_idx(srcw_v, g + 1, b * 64, sidx[b])
                pltpu.async_copy(y_hbm.at[sidx[b]], bufs[b], gsem[b])
        return carry

    lax.fori_loop(0, CHUNKS2 // 2, _body, 0)
    plsc.subcore_barrier()
    pltpu.sync_copy(acc_sh.at[pl.ds(s * STRIPE, 128)], buf0)
    for k in range(STRIPE // 128):
        bb = bufs[k % 2]
        cp = pltpu.async_copy(
            bb, out_hbm.at[c, pl.ds(s * STRIPE + k * 128, 128)], gsem[k % 2])
        if k + 1 < STRIPE // 128:
            pltpu.sync_copy(
                acc_sh.at[pl.ds(s * STRIPE + (k + 1) * 128, 128)],
                bufs[(k + 1) % 2])
        cp.wait()


@functools.partial(
    pl.kernel,
    out_type=jax.ShapeDtypeStruct((NP,), jnp.float32),
    mesh=_mesh,
    scratch_types=[
        pltpu.VMEM((CHUNKS1, 128), jnp.int32),
        pltpu.VMEM((CHUNKS1, 128), jnp.int32),
        pltpu.VMEM((128,), jnp.float32),
        pltpu.VMEM((128,), jnp.float32),
        pltpu.VMEM((STRIPE,), jnp.float32),
        pltpu.VMEM((STRIPE,), jnp.float32),
        pltpu.VMEM((STRIPE,), jnp.float32),
        pltpu.VMEM((STRIPE,), jnp.float32),
        pltpu.VMEM((16,), jnp.float32),
        pltpu.VMEM_SHARED((NP,), jnp.float32),
        pltpu.VMEM_SHARED((NP,), jnp.float32),
        pltpu.SemaphoreType.DMA,
        pltpu.SemaphoreType.DMA,
    ],
)
def _sc_scalar_agg(src_hbm, dst_hbm, t_hbm, dis_hbm, b2_hbm, out_hbm,
                   src_v, dst_v, vals0, vals1, av, dv, tv, ov, b2_v,
                   t_sh, acc_sh, sem0, sem1):
    c = lax.axis_index("c")
    s = lax.axis_index("s")

    @pl.when(c == 0)
    def _():
        pltpu.sync_copy(src_hbm.at[pl.ds(s * CHUNKS1, CHUNKS1)], src_v)
        pltpu.sync_copy(dst_hbm.at[pl.ds(s * CHUNKS1, CHUNKS1)], dst_v)
        pltpu.sync_copy(b2_hbm, b2_v)
        pltpu.sync_copy(t_hbm.at[pl.ds(s * STRIPE, STRIPE)], tv)
        pltpu.sync_copy(tv, t_sh.at[pl.ds(s * STRIPE, STRIPE)])
        zero16 = jnp.zeros((16,), jnp.float32)

        def _zero(i, carry):
            av[pl.ds(i * 16, 16)] = zero16
            return carry

        lax.fori_loop(0, STRIPE // 16, _zero, 0)
        pltpu.sync_copy(av, acc_sh.at[pl.ds(s * STRIPE, STRIPE)])
        plsc.subcore_barrier()

        vals = (vals0, vals1)
        sems = (sem0, sem1)
        for b in range(2):
            pltpu.async_copy(t_sh.at[src_v.at[b]], vals[b], sems[b])

        def _scatter(g, carry):
            for b in range(2):
                ch = g * 2 + b
                pltpu.make_async_copy(
                    t_sh.at[src_v.at[ch]], vals[b], sems[b]).wait()
                pltpu.sync_copy(vals[b], acc_sh.at[dst_v.at[ch]], add=True)

                @pl.when(ch + 2 < CHUNKS1)
                def _():
                    pltpu.async_copy(t_sh.at[src_v.at[ch + 2]], vals[b], sems[b])
            return carry

        lax.fori_loop(0, CHUNKS1 // 2, _scatter, 0)
        plsc.subcore_barrier()
        pltpu.sync_copy(acc_sh.at[pl.ds(s * STRIPE, STRIPE)], av)
        pltpu.sync_copy(dis_hbm.at[pl.ds(s * STRIPE, STRIPE)], dv)
        b2r = b2_v[...]

        def _combine(i, carry):
            a16 = av[pl.ds(i * 16, 16)]
            d16 = dv[pl.ds(i * 16, 16)]
            t16 = tv[pl.ds(i * 16, 16)]
            ov[pl.ds(i * 16, 16)] = d16 * (a16 + t16) + b2r
            return carry

        lax.fori_loop(0, STRIPE // 16, _combine, 0)
        pltpu.sync_copy(ov, out_hbm.at[pl.ds(s * STRIPE, STRIPE)])


def _tc1a_body(x_ref, w_ref, xw_ref):
    xw_ref[...] = jnp.dot(x_ref[...], w_ref[...],
                          preferred_element_type=jnp.float32)


def _tc1a(xp, W1):
    blk = 1024
    grid = NP // blk
    return pl.pallas_call(
        _tc1a_body,
        grid=(grid,),
        in_specs=[
            pl.BlockSpec((blk, 128), lambda i: (i, 0)),
            pl.BlockSpec((128, 128), lambda i: (0, 0)),
        ],
        out_specs=pl.BlockSpec((blk, 128), lambda i: (i, 0)),
        out_shape=jax.ShapeDtypeStruct((NP, 128), jnp.float32),
    )(xp, W1)


def _tc1b_body(xw_ref, dg_ref, y_ref, dis_ref):
    deg = 1.0 + dg_ref[:, 0:1] + dg_ref[:, 1:2]
    dis = lax.rsqrt(deg)
    y_ref[...] = xw_ref[...] * dis
    dis_ref[...] = dis


def _tc1b(xw, degp_t):
    blk = 1024
    grid = NP // blk
    return pl.pallas_call(
        _tc1b_body,
        grid=(grid,),
        in_specs=[
            pl.BlockSpec((blk, 128), lambda i: (i, 0)),
            pl.BlockSpec((blk, 2), lambda i: (i, 0)),
        ],
        out_specs=[
            pl.BlockSpec((blk, 128), lambda i: (i, 0)),
            pl.BlockSpec((blk, 1), lambda i: (i, 0)),
        ],
        out_shape=[
            jax.ShapeDtypeStruct((NP, 128), jnp.float32),
            jax.ShapeDtypeStruct((NP, 1), jnp.float32),
        ],
    )(xw, degp_t)


def _tc2_body(ap_ref, y_ref, dis_ref, b1_ref, w2_ref, t_ref):
    agg = ap_ref[0] + ap_ref[1] + y_ref[...]
    h = jnp.maximum(agg * dis_ref[...] + b1_ref[...], 0.0)
    srow = jnp.sum(h * w2_ref[...], axis=1, keepdims=True)
    t_ref[...] = srow * dis_ref[...]


def _tc2(aggp, y, dis, b1r, w2r):
    blk = 1024
    grid = NP // blk
    return pl.pallas_call(
        _tc2_body,
        grid=(grid,),
        in_specs=[
            pl.BlockSpec((2, blk, 128), lambda i: (0, i, 0)),
            pl.BlockSpec((blk, 128), lambda i: (i, 0)),
            pl.BlockSpec((blk, 1), lambda i: (i, 0)),
            pl.BlockSpec((1, 128), lambda i: (0, 0)),
            pl.BlockSpec((1, 128), lambda i: (0, 0)),
        ],
        out_specs=pl.BlockSpec((blk, 1), lambda i: (i, 0)),
        out_shape=jax.ShapeDtypeStruct((NP, 1), jnp.float32),
    )(aggp, y, dis, b1r, w2r)


def _pack2x16(idx2d):
    w = idx2d[:, :64] | (idx2d[:, 64:] << 16)
    return w.reshape(EROWS // 2, 128)


@jax.jit
def kernel(x, edge_index, W1, b1, W2, b2):
    pad = (jnp.arange(EP - E, dtype=jnp.int32) % (NP - N)) + N
    srcp = jnp.concatenate([edge_index[0], pad]).reshape(EROWS, 128)
    dstp = jnp.concatenate([edge_index[1], pad]).reshape(EROWS, 128)
    srcw = _pack2x16(srcp)
    dstw = _pack2x16(dstp)
    xp = jnp.pad(x, ((0, NP - N), (0, 0)))

    degp = _sc_degree(dstp)
    xw = _tc1a(xp, W1)
    y, dis = _tc1b(xw, jnp.transpose(degp))
    aggp = _sc_row_agg(srcw, dstw, y)
    t = _tc2(aggp, y, dis, b1.reshape(1, 128), W2.reshape(1, 128))
    out_full = _sc_scalar_agg(
        srcp, dstp, t.reshape(-1), dis.reshape(-1),
        jnp.broadcast_to(b2, (16,)))
    return out_full[:N]


# speedup vs baseline: 1.1247x; 1.0264x over previous
import functools

import jax
import jax.numpy as jnp
from jax import lax
from jax.experimental import pallas as pl
from jax.experimental.pallas import tpu as pltpu
from jax.experimental.pallas import tpu_sc as plsc

N = 10000
NP = 10240
E = 320000
EP = 327680
EROWS = EP // 128
CHUNKS2 = EROWS // 32
CHUNKS1 = EROWS // 16
STRIPE = NP // 16

_mesh = plsc.VectorSubcoreMesh(core_axis_name="c", subcore_axis_name="s")


@functools.partial(
    pl.kernel,
    out_type=jax.ShapeDtypeStruct((2, NP), jnp.float32),
    mesh=_mesh,
    scratch_types=[
        pltpu.VMEM((CHUNKS2, 128), jnp.int32),
        pltpu.VMEM((128,), jnp.float32),
        pltpu.VMEM((STRIPE,), jnp.float32),
        pltpu.VMEM_SHARED((NP,), jnp.float32),
    ],
)
def _sc_degree(dst_hbm, out_hbm, dst_v, ones_v, st_v, acc_sh):
    c = lax.axis_index("c")
    s = lax.axis_index("s")
    tile = c * 16 + s
    pltpu.sync_copy(dst_hbm.at[pl.ds(tile * CHUNKS2, CHUNKS2)], dst_v)
    one16 = jnp.full((16,), 1.0, jnp.float32)
    zero16 = jnp.zeros((16,), jnp.float32)
    for k in range(8):
        ones_v[pl.ds(k * 16, 16)] = one16

    def _zero(i, carry):
        st_v[pl.ds(i * 16, 16)] = zero16
        return carry

    lax.fori_loop(0, STRIPE // 16, _zero, 0)
    pltpu.sync_copy(st_v, acc_sh.at[pl.ds(s * STRIPE, STRIPE)])
    plsc.subcore_barrier()

    def _scatter(ch, carry):
        pltpu.sync_copy(ones_v, acc_sh.at[dst_v.at[ch]], add=True)
        return carry

    lax.fori_loop(0, CHUNKS2, _scatter, 0)
    plsc.subcore_barrier()
    pltpu.sync_copy(acc_sh.at[pl.ds(s * STRIPE, STRIPE)], st_v)
    pltpu.sync_copy(st_v, out_hbm.at[c, pl.ds(s * STRIPE, STRIPE)])


def _unpack_idx(wv, row, colbase, out_v):
    for j in range(4):
        w16 = wv[row, pl.ds(colbase + j * 16, 16)]
        out_v[pl.ds(j * 16, 16)] = w16 & 0xFFFF
        out_v[pl.ds(64 + j * 16, 16)] = lax.shift_right_logical(w16, 16)


@functools.partial(
    pl.kernel,
    out_type=jax.ShapeDtypeStruct((2, NP, 128), jnp.float32),
    mesh=_mesh,
    scratch_types=[
        pltpu.VMEM((CHUNKS2 // 2, 128), jnp.int32),
        pltpu.VMEM((CHUNKS2 // 2, 128), jnp.int32),
        pltpu.VMEM((128,), jnp.int32),
        pltpu.VMEM((128,), jnp.int32),
        pltpu.VMEM((128,), jnp.int32),
        pltpu.VMEM((128,), jnp.int32),
        pltpu.VMEM((128, 128), jnp.float32),
        pltpu.VMEM((128, 128), jnp.float32),
        pltpu.VMEM_SHARED((NP, 128), jnp.float32),
        pltpu.SemaphoreType.DMA,
        pltpu.SemaphoreType.DMA,
        pltpu.SemaphoreType.DMA,
        pltpu.SemaphoreType.DMA,
    ],
)
def _sc_row_agg(srcw_hbm, dstw_hbm, y_hbm, out_hbm,
                srcw_v, dstw_v, sidx0, sidx1, didx0, didx1,
                buf0, buf1, acc_sh, gsem0, gsem1, ssem0, ssem1):
    c = lax.axis_index("c")
    s = lax.axis_index("s")
    tile = c * 16 + s
    nrows = CHUNKS2 // 2
    pltpu.sync_copy(srcw_hbm.at[pl.ds(tile * nrows, nrows)], srcw_v)
    pltpu.sync_copy(dstw_hbm.at[pl.ds(tile * nrows, nrows)], dstw_v)

    zero16 = jnp.zeros((16,), jnp.float32)

    def _zero(i, carry):
        for k in range(8):
            buf0[i, pl.ds(k * 16, 16)] = zero16
        return carry

    lax.fori_loop(0, 128, _zero, 0)
    for k in range(STRIPE // 128):
        pltpu.sync_copy(buf0, acc_sh.at[pl.ds(s * STRIPE + k * 128, 128)])
    plsc.subcore_barrier()

    bufs = (buf0, buf1)
    gsem = (gsem0, gsem1)
    sidx = (sidx0, sidx1)
    didx = (didx0, didx1)
    for b in range(2):
        _unpack_idx(srcw_v, 0, b * 64, sidx[b])
        pltpu.async_copy(y_hbm.at[sidx[b]], bufs[b], gsem[b])

    def _body(g, carry):
        for b in range(2):
            ch = g * 2 + b
            pltpu.make_async_copy(y_hbm.at[sidx[b]], bufs[b], gsem[b]).wait()
            _unpack_idx(dstw_v, g, b * 64, didx[b])
            pltpu.sync_copy(bufs[b], acc_sh.at[didx[b]], add=True)

            @pl.when(ch + 2 < CHUNKS2)
            def _():
                _unpack_idx(srcw_v, g + 1, b * 64, sidx[b])
                pltpu.async_copy(y_hbm.at[sidx[b]], bufs[b], gsem[b])
        return carry

    lax.fori_loop(0, CHUNKS2 // 2, _body, 0)
    plsc.subcore_barrier()
    pltpu.sync_copy(acc_sh.at[pl.ds(s * STRIPE, 128)], buf0)
    for k in range(STRIPE // 128):
        bb = bufs[k % 2]
        cp = pltpu.async_copy(
            bb, out_hbm.at[c, pl.ds(s * STRIPE + k * 128, 128)], gsem[k % 2])
        if k + 1 < STRIPE // 128:
            pltpu.sync_copy(
                acc_sh.at[pl.ds(s * STRIPE + (k + 1) * 128, 128)],
                bufs[(k + 1) % 2])
        cp.wait()


@functools.partial(
    pl.kernel,
    out_type=jax.ShapeDtypeStruct((NP,), jnp.float32),
    mesh=_mesh,
    scratch_types=[
        pltpu.VMEM((CHUNKS1, 128), jnp.int32),
        pltpu.VMEM((CHUNKS1, 128), jnp.int32),
        pltpu.VMEM((128,), jnp.float32),
        pltpu.VMEM((128,), jnp.float32),
        pltpu.VMEM((STRIPE,), jnp.float32),
        pltpu.VMEM((STRIPE,), jnp.float32),
        pltpu.VMEM((STRIPE,), jnp.float32),
        pltpu.VMEM((STRIPE,), jnp.float32),
        pltpu.VMEM((16,), jnp.float32),
        pltpu.VMEM_SHARED((NP,), jnp.float32),
        pltpu.VMEM_SHARED((NP,), jnp.float32),
        pltpu.SemaphoreType.DMA,
        pltpu.SemaphoreType.DMA,
    ],
)
def _sc_scalar_agg(src_hbm, dst_hbm, t_hbm, dis_hbm, b2_hbm, out_hbm,
                   src_v, dst_v, vals0, vals1, av, dv, tv, ov, b2_v,
                   t_sh, acc_sh, sem0, sem1):
    c = lax.axis_index("c")
    s = lax.axis_index("s")

    @pl.when(c == 0)
    def _():
        pltpu.sync_copy(src_hbm.at[pl.ds(s * CHUNKS1, CHUNKS1)], src_v)
        pltpu.sync_copy(dst_hbm.at[pl.ds(s * CHUNKS1, CHUNKS1)], dst_v)
        pltpu.sync_copy(b2_hbm, b2_v)
        pltpu.sync_copy(t_hbm.at[pl.ds(s * STRIPE, STRIPE)], tv)
        pltpu.sync_copy(tv, t_sh.at[pl.ds(s * STRIPE, STRIPE)])
        zero16 = jnp.zeros((16,), jnp.float32)

        def _zero(i, carry):
            av[pl.ds(i * 16, 16)] = zero16
            return carry

        lax.fori_loop(0, STRIPE // 16, _zero, 0)
        pltpu.sync_copy(av, acc_sh.at[pl.ds(s * STRIPE, STRIPE)])
        plsc.subcore_barrier()

        vals = (vals0, vals1)
        sems = (sem0, sem1)
        for b in range(2):
            pltpu.async_copy(t_sh.at[src_v.at[b]], vals[b], sems[b])

        def _scatter(g, carry):
            for b in range(2):
                ch = g * 2 + b
                pltpu.make_async_copy(
                    t_sh.at[src_v.at[ch]], vals[b], sems[b]).wait()
                pltpu.sync_copy(vals[b], acc_sh.at[dst_v.at[ch]], add=True)

                @pl.when(ch + 2 < CHUNKS1)
                def _():
                    pltpu.async_copy(t_sh.at[src_v.at[ch + 2]], vals[b], sems[b])
            return carry

        lax.fori_loop(0, CHUNKS1 // 2, _scatter, 0)
        plsc.subcore_barrier()
        pltpu.sync_copy(acc_sh.at[pl.ds(s * STRIPE, STRIPE)], av)
        pltpu.sync_copy(dis_hbm.at[pl.ds(s * STRIPE, STRIPE)], dv)
        b2r = b2_v[...]

        def _combine(i, carry):
            a16 = av[pl.ds(i * 16, 16)]
            d16 = dv[pl.ds(i * 16, 16)]
            t16 = tv[pl.ds(i * 16, 16)]
            ov[pl.ds(i * 16, 16)] = d16 * (a16 + t16) + b2r
            return carry

        lax.fori_loop(0, STRIPE // 16, _combine, 0)
        pltpu.sync_copy(ov, out_hbm.at[pl.ds(s * STRIPE, STRIPE)])


def _tc1_body(x_ref, w_ref, dg_ref, y_ref, dis_ref):
    deg = 1.0 + dg_ref[:, 0:1] + dg_ref[:, 1:2]
    dis = lax.rsqrt(deg)
    xw = jnp.dot(x_ref[...], w_ref[...], preferred_element_type=jnp.float32)
    y_ref[...] = xw * dis
    dis_ref[...] = dis


def _tc1(xp, W1, degp_t):
    blk = 1024
    grid = NP // blk
    return pl.pallas_call(
        _tc1_body,
        grid=(grid,),
        in_specs=[
            pl.BlockSpec((blk, 128), lambda i: (i, 0)),
            pl.BlockSpec((128, 128), lambda i: (0, 0)),
            pl.BlockSpec((blk, 2), lambda i: (i, 0)),
        ],
        out_specs=[
            pl.BlockSpec((blk, 128), lambda i: (i, 0)),
            pl.BlockSpec((blk, 1), lambda i: (i, 0)),
        ],
        out_shape=[
            jax.ShapeDtypeStruct((NP, 128), jnp.float32),
            jax.ShapeDtypeStruct((NP, 1), jnp.float32),
        ],
    )(xp, W1, degp_t)


def _tc2_body(ap_ref, y_ref, dis_ref, b1_ref, w2_ref, t_ref):
    agg = ap_ref[0] + ap_ref[1] + y_ref[...]
    h = jnp.maximum(agg * dis_ref[...] + b1_ref[...], 0.0)
    srow = jnp.sum(h * w2_ref[...], axis=1, keepdims=True)
    t_ref[...] = srow * dis_ref[...]


def _tc2(aggp, y, dis, b1r, w2r):
    blk = 1024
    grid = NP // blk
    return pl.pallas_call(
        _tc2_body,
        grid=(grid,),
        in_specs=[
            pl.BlockSpec((2, blk, 128), lambda i: (0, i, 0)),
            pl.BlockSpec((blk, 128), lambda i: (i, 0)),
            pl.BlockSpec((blk, 1), lambda i: (i, 0)),
            pl.BlockSpec((1, 128), lambda i: (0, 0)),
            pl.BlockSpec((1, 128), lambda i: (0, 0)),
        ],
        out_specs=pl.BlockSpec((blk, 1), lambda i: (i, 0)),
        out_shape=jax.ShapeDtypeStruct((NP, 1), jnp.float32),
    )(aggp, y, dis, b1r, w2r)


def _pack2x16(idx2d):
    w = idx2d[:, :64] | (idx2d[:, 64:] << 16)
    return w.reshape(EROWS // 2, 128)


@jax.jit
def kernel(x, edge_index, W1, b1, W2, b2):
    pad = (jnp.arange(EP - E, dtype=jnp.int32) % (NP - N)) + N
    srcp = jnp.concatenate([edge_index[0], pad]).reshape(EROWS, 128)
    dstp = jnp.concatenate([edge_index[1], pad]).reshape(EROWS, 128)
    srcw = _pack2x16(srcp)
    dstw = _pack2x16(dstp)
    xp = jnp.pad(x, ((0, NP - N), (0, 0)))

    degp = _sc_degree(dstp)
    y, dis = _tc1(xp, W1, jnp.transpose(degp))
    aggp = _sc_row_agg(srcw, dstw, y)
    t = _tc2(aggp, y, dis, b1.reshape(1, 128), W2.reshape(1, 128))
    out_full = _sc_scalar_agg(
        srcp, dstp, t.reshape(-1), dis.reshape(-1),
        jnp.broadcast_to(b2, (16,)))
    return out_full[:N]


# speedup vs baseline: 1.1806x; 1.0497x over previous
import functools

import jax
import jax.numpy as jnp
from jax import lax
from jax.experimental import pallas as pl
from jax.experimental.pallas import tpu as pltpu
from jax.experimental.pallas import tpu_sc as plsc

N = 10000
NP = 10240
E = 320000
EP = 327680
EROWS = EP // 128
CHUNKS2 = EROWS // 32
CHUNKS1 = EROWS // 16
STRIPE = NP // 16

_mesh = plsc.VectorSubcoreMesh(core_axis_name="c", subcore_axis_name="s")


@functools.partial(
    pl.kernel,
    out_type=jax.ShapeDtypeStruct((2, NP), jnp.float32),
    mesh=_mesh,
    scratch_types=[
        pltpu.VMEM((CHUNKS2, 128), jnp.int32),
        pltpu.VMEM((128,), jnp.float32),
        pltpu.VMEM((STRIPE,), jnp.float32),
        pltpu.VMEM_SHARED((NP,), jnp.float32),
    ],
)
def _sc_degree(dst_hbm, out_hbm, dst_v, ones_v, st_v, acc_sh):
    c = lax.axis_index("c")
    s = lax.axis_index("s")
    tile = c * 16 + s
    pltpu.sync_copy(dst_hbm.at[pl.ds(tile * CHUNKS2, CHUNKS2)], dst_v)
    one16 = jnp.full((16,), 1.0, jnp.float32)
    zero16 = jnp.zeros((16,), jnp.float32)
    for k in range(8):
        ones_v[pl.ds(k * 16, 16)] = one16

    def _zero(i, carry):
        st_v[pl.ds(i * 16, 16)] = zero16
        return carry

    lax.fori_loop(0, STRIPE // 16, _zero, 0)
    pltpu.sync_copy(st_v, acc_sh.at[pl.ds(s * STRIPE, STRIPE)])
    plsc.subcore_barrier()

    def _scatter(ch, carry):
        pltpu.sync_copy(ones_v, acc_sh.at[dst_v.at[ch]], add=True)
        return carry

    lax.fori_loop(0, CHUNKS2, _scatter, 0)
    plsc.subcore_barrier()
    pltpu.sync_copy(acc_sh.at[pl.ds(s * STRIPE, STRIPE)], st_v)
    pltpu.sync_copy(st_v, out_hbm.at[c, pl.ds(s * STRIPE, STRIPE)])


def _unpack_idx(wv, row, colbase, out_v):
    for j in range(4):
        w16 = wv[row, pl.ds(colbase + j * 16, 16)]
        out_v[pl.ds(j * 16, 16)] = w16 & 0xFFFF
        out_v[pl.ds(64 + j * 16, 16)] = lax.shift_right_logical(w16, 16)


@functools.partial(
    pl.kernel,
    out_type=jax.ShapeDtypeStruct((2, NP, 128), jnp.float32),
    mesh=_mesh,
    scratch_types=[
        pltpu.VMEM((CHUNKS2 // 2, 128), jnp.int32),
        pltpu.VMEM((CHUNKS2 // 2, 128), jnp.int32),
        pltpu.VMEM((128,), jnp.int32),
        pltpu.VMEM((128,), jnp.int32),
        pltpu.VMEM((128,), jnp.int32),
        pltpu.VMEM((128,), jnp.int32),
        pltpu.VMEM((128, 128), jnp.float32),
        pltpu.VMEM((128, 128), jnp.float32),
        pltpu.VMEM_SHARED((NP, 128), jnp.float32),
        pltpu.SemaphoreType.DMA,
        pltpu.SemaphoreType.DMA,
        pltpu.SemaphoreType.DMA,
        pltpu.SemaphoreType.DMA,
    ],
)
def _sc_row_agg(srcw_hbm, dstw_hbm, y_hbm, out_hbm,
                srcw_v, dstw_v, sidx0, sidx1, didx0, didx1,
                buf0, buf1, acc_sh, gsem0, gsem1, ssem0, ssem1):
    c = lax.axis_index("c")
    s = lax.axis_index("s")
    tile = c * 16 + s
    nrows = CHUNKS2 // 2
    cps = pltpu.async_copy(
        srcw_hbm.at[pl.ds(tile * nrows, nrows)], srcw_v, ssem0)
    cpd = pltpu.async_copy(
        dstw_hbm.at[pl.ds(tile * nrows, nrows)], dstw_v, ssem1)

    zero16 = jnp.zeros((16,), jnp.float32)

    def _zero(i, carry):
        for k in range(8):
            buf0[i, pl.ds(k * 16, 16)] = zero16
        return carry

    lax.fori_loop(0, 128, _zero, 0)
    zcps = [
        pltpu.async_copy(
            buf0, acc_sh.at[pl.ds(s * STRIPE + k * 128, 128)], gsem0)
        for k in range(STRIPE // 128)
    ]
    for cp in zcps:
        cp.wait()
    cps.wait()
    cpd.wait()
    plsc.subcore_barrier()

    bufs = (buf0, buf1)
    gsem = (gsem0, gsem1)
    sidx = (sidx0, sidx1)
    didx = (didx0, didx1)
    for b in range(2):
        _unpack_idx(srcw_v, 0, b * 64, sidx[b])
        pltpu.async_copy(y_hbm.at[sidx[b]], bufs[b], gsem[b])

    def _body(g, carry):
        for b in range(2):
            ch = g * 2 + b
            pltpu.make_async_copy(y_hbm.at[sidx[b]], bufs[b], gsem[b]).wait()
            _unpack_idx(dstw_v, g, b * 64, didx[b])
            pltpu.sync_copy(bufs[b], acc_sh.at[didx[b]], add=True)

            @pl.when(ch + 2 < CHUNKS2)
            def _():
                _unpack_idx(srcw_v, g + 1, b * 64, sidx[b])
                pltpu.async_copy(y_hbm.at[sidx[b]], bufs[b], gsem[b])
        return carry

    lax.fori_loop(0, CHUNKS2 // 2, _body, 0)
    plsc.subcore_barrier()
    pltpu.sync_copy(acc_sh.at[pl.ds(s * STRIPE, 128)], buf0)
    for k in range(STRIPE // 128):
        bb = bufs[k % 2]
        cp = pltpu.async_copy(
            bb, out_hbm.at[c, pl.ds(s * STRIPE + k * 128, 128)], gsem[k % 2])
        if k + 1 < STRIPE // 128:
            pltpu.sync_copy(
                acc_sh.at[pl.ds(s * STRIPE + (k + 1) * 128, 128)],
                bufs[(k + 1) % 2])
        cp.wait()


@functools.partial(
    pl.kernel,
    out_type=jax.ShapeDtypeStruct((NP,), jnp.float32),
    mesh=_mesh,
    scratch_types=[
        pltpu.VMEM((CHUNKS1, 128), jnp.int32),
        pltpu.VMEM((CHUNKS1, 128), jnp.int32),
        pltpu.VMEM((128,), jnp.float32),
        pltpu.VMEM((128,), jnp.float32),
        pltpu.VMEM((128,), jnp.float32),
        pltpu.VMEM((128,), jnp.float32),
        pltpu.VMEM((STRIPE,), jnp.float32),
        pltpu.VMEM((STRIPE,), jnp.float32),
        pltpu.VMEM((STRIPE,), jnp.float32),
        pltpu.VMEM((STRIPE,), jnp.float32),
        pltpu.VMEM((16,), jnp.float32),
        pltpu.VMEM_SHARED((NP,), jnp.float32),
        pltpu.VMEM_SHARED((NP,), jnp.float32),
        [pltpu.SemaphoreType.DMA] * 4,
        [pltpu.SemaphoreType.DMA] * 4,
    ],
)
def _sc_scalar_agg(src_hbm, dst_hbm, t_hbm, dis_hbm, b2_hbm, out_hbm,
                   src_v, dst_v, vals0, vals1, vals2, vals3, av, dv, tv, ov,
                   b2_v, t_sh, acc_sh, gs, ss):
    c = lax.axis_index("c")
    s = lax.axis_index("s")

    @pl.when(c == 0)
    def _():
        pltpu.sync_copy(src_hbm.at[pl.ds(s * CHUNKS1, CHUNKS1)], src_v)
        pltpu.sync_copy(dst_hbm.at[pl.ds(s * CHUNKS1, CHUNKS1)], dst_v)
        pltpu.sync_copy(b2_hbm, b2_v)
        pltpu.sync_copy(t_hbm.at[pl.ds(s * STRIPE, STRIPE)], tv)
        pltpu.sync_copy(tv, t_sh.at[pl.ds(s * STRIPE, STRIPE)])
        zero16 = jnp.zeros((16,), jnp.float32)

        def _zero(i, carry):
            av[pl.ds(i * 16, 16)] = zero16
            return carry

        lax.fori_loop(0, STRIPE // 16, _zero, 0)
        pltpu.sync_copy(av, acc_sh.at[pl.ds(s * STRIPE, STRIPE)])
        plsc.subcore_barrier()

        vals = (vals0, vals1, vals2, vals3)
        for b in range(2):
            pltpu.async_copy(t_sh.at[src_v.at[b]], vals[b], gs[b])

        def _scatter(g, carry):
            for b in range(4):
                ch = g * 4 + b
                pltpu.make_async_copy(
                    t_sh.at[src_v.at[ch]], vals[b], gs[b]).wait()
                pltpu.async_copy(
                    vals[b], acc_sh.at[dst_v.at[ch]], ss[b], add=True)
                b2ago = (b - 2) % 4

                @pl.when(ch - 2 >= 0)
                def _():
                    pltpu.make_async_copy(
                        vals[b2ago], acc_sh.at[dst_v.at[ch - 2]],
                        ss[b2ago]).wait()

                @pl.when(ch + 2 < CHUNKS1)
                def _():
                    pltpu.async_copy(
                        t_sh.at[src_v.at[ch + 2]], vals[b2ago], gs[b2ago])
            return carry

        lax.fori_loop(0, CHUNKS1 // 4, _scatter, 0)
        pltpu.make_async_copy(
            vals[2], acc_sh.at[dst_v.at[CHUNKS1 - 2]], ss[2]).wait()
        pltpu.make_async_copy(
            vals[3], acc_sh.at[dst_v.at[CHUNKS1 - 1]], ss[3]).wait()
        plsc.subcore_barrier()
        pltpu.sync_copy(acc_sh.at[pl.ds(s * STRIPE, STRIPE)], av)
        pltpu.sync_copy(dis_hbm.at[pl.ds(s * STRIPE, STRIPE)], dv)
        b2r = b2_v[...]

        def _combine(i, carry):
            a16 = av[pl.ds(i * 16, 16)]
            d16 = dv[pl.ds(i * 16, 16)]
            t16 = tv[pl.ds(i * 16, 16)]
            ov[pl.ds(i * 16, 16)] = d16 * (a16 + t16) + b2r
            return carry

        lax.fori_loop(0, STRIPE // 16, _combine, 0)
        pltpu.sync_copy(ov, out_hbm.at[pl.ds(s * STRIPE, STRIPE)])


def _tc1_body(x_ref, w_ref, dg_ref, y_ref, dis_ref):
    deg = 1.0 + dg_ref[:, 0:1] + dg_ref[:, 1:2]
    dis = lax.rsqrt(deg)
    xw = jnp.dot(x_ref[...], w_ref[...], preferred_element_type=jnp.float32)
    y_ref[...] = xw * dis
    dis_ref[...] = dis


def _tc1(xp, W1, degp_t):
    blk = 1024
    grid = NP // blk
    return pl.pallas_call(
        _tc1_body,
        grid=(grid,),
        in_specs=[
            pl.BlockSpec((blk, 128), lambda i: (i, 0)),
            pl.BlockSpec((128, 128), lambda i: (0, 0)),
            pl.BlockSpec((blk, 2), lambda i: (i, 0)),
        ],
        out_specs=[
            pl.BlockSpec((blk, 128), lambda i: (i, 0)),
            pl.BlockSpec((blk, 1), lambda i: (i, 0)),
        ],
        out_shape=[
            jax.ShapeDtypeStruct((NP, 128), jnp.float32),
            jax.ShapeDtypeStruct((NP, 1), jnp.float32),
        ],
    )(xp, W1, degp_t)


def _tc2_body(ap_ref, y_ref, dis_ref, b1_ref, w2_ref, t_ref):
    agg = ap_ref[0] + ap_ref[1] + y_ref[...]
    h = jnp.maximum(agg * dis_ref[...] + b1_ref[...], 0.0)
    srow = jnp.sum(h * w2_ref[...], axis=1, keepdims=True)
    t_ref[...] = srow * dis_ref[...]


def _tc2(aggp, y, dis, b1r, w2r):
    blk = 1024
    grid = NP // blk
    return pl.pallas_call(
        _tc2_body,
        grid=(grid,),
        in_specs=[
            pl.BlockSpec((2, blk, 128), lambda i: (0, i, 0)),
            pl.BlockSpec((blk, 128), lambda i: (i, 0)),
            pl.BlockSpec((blk, 1), lambda i: (i, 0)),
            pl.BlockSpec((1, 128), lambda i: (0, 0)),
            pl.BlockSpec((1, 128), lambda i: (0, 0)),
        ],
        out_specs=pl.BlockSpec((blk, 1), lambda i: (i, 0)),
        out_shape=jax.ShapeDtypeStruct((NP, 1), jnp.float32),
    )(aggp, y, dis, b1r, w2r)


def _pack2x16(idx2d):
    w = idx2d[:, :64] | (idx2d[:, 64:] << 16)
    return w.reshape(EROWS // 2, 128)


@jax.jit
def kernel(x, edge_index, W1, b1, W2, b2):
    pad = (jnp.arange(EP - E, dtype=jnp.int32) % (NP - N)) + N
    srcp = jnp.concatenate([edge_index[0], pad]).reshape(EROWS, 128)
    dstp = jnp.concatenate([edge_index[1], pad]).reshape(EROWS, 128)
    srcw = _pack2x16(srcp)
    dstw = _pack2x16(dstp)
    xp = jnp.pad(x, ((0, NP - N), (0, 0)))

    degp = _sc_degree(dstp)
    y, dis = _tc1(xp, W1, jnp.transpose(degp))
    aggp = _sc_row_agg(srcw, dstw, y)
    t = _tc2(aggp, y, dis, b1.reshape(1, 128), W2.reshape(1, 128))
    out_full = _sc_scalar_agg(
        srcp, dstp, t.reshape(-1), dis.reshape(-1),
        jnp.broadcast_to(b2, (16,)))
    return out_full[:N]


# speedup vs baseline: 1.2105x; 1.0253x over previous
import functools

import jax
import jax.numpy as jnp
from jax import lax
from jax.experimental import pallas as pl
from jax.experimental.pallas import tpu as pltpu
from jax.experimental.pallas import tpu_sc as plsc

N = 10000
NP = 10240
E = 320000
EP = 327680
EROWS = EP // 128
CHUNKS2 = EROWS // 32
CHUNKS1 = EROWS // 16
STRIPE = NP // 16

_mesh = plsc.VectorSubcoreMesh(core_axis_name="c", subcore_axis_name="s")


@functools.partial(
    pl.kernel,
    out_type=jax.ShapeDtypeStruct((2, NP), jnp.float32),
    mesh=_mesh,
    scratch_types=[
        pltpu.VMEM((CHUNKS2, 128), jnp.int32),
        pltpu.VMEM((128,), jnp.float32),
        pltpu.VMEM((STRIPE,), jnp.float32),
        pltpu.VMEM_SHARED((NP,), jnp.float32),
        [pltpu.SemaphoreType.DMA] * 4,
    ],
)
def _sc_degree(dst_hbm, out_hbm, dst_v, ones_v, st_v, acc_sh, ss):
    c = lax.axis_index("c")
    s = lax.axis_index("s")
    tile = c * 16 + s
    cpd = pltpu.async_copy(
        dst_hbm.at[pl.ds(tile * CHUNKS2, CHUNKS2)], dst_v, ss[0])
    one16 = jnp.full((16,), 1.0, jnp.float32)
    zero16 = jnp.zeros((16,), jnp.float32)
    for k in range(8):
        ones_v[pl.ds(k * 16, 16)] = one16

    def _zero(i, carry):
        st_v[pl.ds(i * 16, 16)] = zero16
        return carry

    lax.fori_loop(0, STRIPE // 16, _zero, 0)
    pltpu.sync_copy(st_v, acc_sh.at[pl.ds(s * STRIPE, STRIPE)])
    cpd.wait()
    plsc.subcore_barrier()

    def _scatter(g, carry):
        for b in range(4):
            ch = g * 4 + b

            @pl.when(ch - 4 >= 0)
            def _():
                pltpu.make_async_copy(
                    ones_v, acc_sh.at[dst_v.at[ch - 4]], ss[b]).wait()

            pltpu.async_copy(ones_v, acc_sh.at[dst_v.at[ch]], ss[b], add=True)
        return carry

    lax.fori_loop(0, CHUNKS2 // 4, _scatter, 0)
    for b in range(4):
        ch = CHUNKS2 - 4 + b
        pltpu.make_async_copy(ones_v, acc_sh.at[dst_v.at[ch]], ss[b]).wait()
    plsc.subcore_barrier()
    pltpu.sync_copy(acc_sh.at[pl.ds(s * STRIPE, STRIPE)], st_v)
    pltpu.sync_copy(st_v, out_hbm.at[c, pl.ds(s * STRIPE, STRIPE)])


def _unpack_idx(wv, row, colbase, out_v):
    for j in range(4):
        w16 = wv[row, pl.ds(colbase + j * 16, 16)]
        out_v[pl.ds(j * 16, 16)] = w16 & 0xFFFF
        out_v[pl.ds(64 + j * 16, 16)] = lax.shift_right_logical(w16, 16)


@functools.partial(
    pl.kernel,
    out_type=jax.ShapeDtypeStruct((2, NP, 128), jnp.float32),
    mesh=_mesh,
    scratch_types=[
        pltpu.VMEM((CHUNKS2 // 2, 128), jnp.int32),
        pltpu.VMEM((CHUNKS2 // 2, 128), jnp.int32),
        pltpu.VMEM((128,), jnp.int32),
        pltpu.VMEM((128,), jnp.int32),
        pltpu.VMEM((128,), jnp.int32),
        pltpu.VMEM((128,), jnp.int32),
        pltpu.VMEM((128, 128), jnp.float32),
        pltpu.VMEM((128, 128), jnp.float32),
        pltpu.VMEM_SHARED((NP, 128), jnp.float32),
        pltpu.SemaphoreType.DMA,
        pltpu.SemaphoreType.DMA,
        pltpu.SemaphoreType.DMA,
        pltpu.SemaphoreType.DMA,
    ],
)
def _sc_row_agg(srcw_hbm, dstw_hbm, y_hbm, out_hbm,
                srcw_v, dstw_v, sidx0, sidx1, didx0, didx1,
                buf0, buf1, acc_sh, gsem0, gsem1, ssem0, ssem1):
    c = lax.axis_index("c")
    s = lax.axis_index("s")
    tile = c * 16 + s
    nrows = CHUNKS2 // 2
    cps = pltpu.async_copy(
        srcw_hbm.at[pl.ds(tile * nrows, nrows)], srcw_v, ssem0)
    cpd = pltpu.async_copy(
        dstw_hbm.at[pl.ds(tile * nrows, nrows)], dstw_v, ssem1)

    zero16 = jnp.zeros((16,), jnp.float32)

    def _zero(i, carry):
        for k in range(8):
            buf0[i, pl.ds(k * 16, 16)] = zero16
        return carry

    lax.fori_loop(0, 128, _zero, 0)
    zcps = [
        pltpu.async_copy(
            buf0, acc_sh.at[pl.ds(s * STRIPE + k * 128, 128)], gsem0)
        for k in range(STRIPE // 128)
    ]
    for cp in zcps:
        cp.wait()
    cps.wait()
    cpd.wait()
    plsc.subcore_barrier()

    bufs = (buf0, buf1)
    gsem = (gsem0, gsem1)
    sidx = (sidx0, sidx1)
    didx = (didx0, didx1)
    for b in range(2):
        _unpack_idx(srcw_v, 0, b * 64, sidx[b])
        pltpu.async_copy(y_hbm.at[sidx[b]], bufs[b], gsem[b])

    def _body(g, carry):
        for b in range(2):
            ch = g * 2 + b
            pltpu.make_async_copy(y_hbm.at[sidx[b]], bufs[b], gsem[b]).wait()
            _unpack_idx(dstw_v, g, b * 64, didx[b])
            pltpu.sync_copy(bufs[b], acc_sh.at[didx[b]], add=True)

            @pl.when(ch + 2 < CHUNKS2)
            def _():
                _unpack_idx(srcw_v, g + 1, b * 64, sidx[b])
                pltpu.async_copy(y_hbm.at[sidx[b]], bufs[b], gsem[b])
        return carry

    lax.fori_loop(0, CHUNKS2 // 2, _body, 0)
    plsc.subcore_barrier()
    pltpu.sync_copy(acc_sh.at[pl.ds(s * STRIPE, 128)], buf0)
    for k in range(STRIPE // 128):
        bb = bufs[k % 2]
        cp = pltpu.async_copy(
            bb, out_hbm.at[c, pl.ds(s * STRIPE + k * 128, 128)], gsem[k % 2])
        if k + 1 < STRIPE // 128:
            pltpu.sync_copy(
                acc_sh.at[pl.ds(s * STRIPE + (k + 1) * 128, 128)],
                bufs[(k + 1) % 2])
        cp.wait()


@functools.partial(
    pl.kernel,
    out_type=jax.ShapeDtypeStruct((NP,), jnp.float32),
    mesh=_mesh,
    scratch_types=[
        pltpu.VMEM((CHUNKS1, 128), jnp.int32),
        pltpu.VMEM((CHUNKS1, 128), jnp.int32),
        pltpu.VMEM((128,), jnp.float32),
        pltpu.VMEM((128,), jnp.float32),
        pltpu.VMEM((128,), jnp.float32),
        pltpu.VMEM((128,), jnp.float32),
        pltpu.VMEM((STRIPE,), jnp.float32),
        pltpu.VMEM((STRIPE,), jnp.float32),
        pltpu.VMEM((STRIPE,), jnp.float32),
        pltpu.VMEM((STRIPE,), jnp.float32),
        pltpu.VMEM((16,), jnp.float32),
        pltpu.VMEM_SHARED((NP,), jnp.float32),
        pltpu.VMEM_SHARED((NP,), jnp.float32),
        [pltpu.SemaphoreType.DMA] * 4,
        [pltpu.SemaphoreType.DMA] * 4,
    ],
)
def _sc_scalar_agg(src_hbm, dst_hbm, t_hbm, dis_hbm, b2_hbm, out_hbm,
                   src_v, dst_v, vals0, vals1, vals2, vals3, av, dv, tv, ov,
                   b2_v, t_sh, acc_sh, gs, ss):
    c = lax.axis_index("c")
    s = lax.axis_index("s")

    @pl.when(c == 0)
    def _():
        cps = pltpu.async_copy(
            src_hbm.at[pl.ds(s * CHUNKS1, CHUNKS1)], src_v, gs[0])
        cpd = pltpu.async_copy(
            dst_hbm.at[pl.ds(s * CHUNKS1, CHUNKS1)], dst_v, gs[1])
        cpt = pltpu.async_copy(t_hbm.at[pl.ds(s * STRIPE, STRIPE)], tv, gs[2])
        pltpu.sync_copy(b2_hbm, b2_v)
        zero16 = jnp.zeros((16,), jnp.float32)

        def _zero(i, carry):
            av[pl.ds(i * 16, 16)] = zero16
            return carry

        lax.fori_loop(0, STRIPE // 16, _zero, 0)
        cpt.wait()
        pltpu.sync_copy(tv, t_sh.at[pl.ds(s * STRIPE, STRIPE)])
        pltpu.sync_copy(av, acc_sh.at[pl.ds(s * STRIPE, STRIPE)])
        cps.wait()
        cpd.wait()
        plsc.subcore_barrier()

        vals = (vals0, vals1, vals2, vals3)
        for b in range(2):
            pltpu.async_copy(t_sh.at[src_v.at[b]], vals[b], gs[b])

        def _scatter(g, carry):
            for b in range(4):
                ch = g * 4 + b
                pltpu.make_async_copy(
                    t_sh.at[src_v.at[ch]], vals[b], gs[b]).wait()
                pltpu.async_copy(
                    vals[b], acc_sh.at[dst_v.at[ch]], ss[b], add=True)
                b2ago = (b - 2) % 4

                @pl.when(ch - 2 >= 0)
                def _():
                    pltpu.make_async_copy(
                        vals[b2ago], acc_sh.at[dst_v.at[ch - 2]],
                        ss[b2ago]).wait()

                @pl.when(ch + 2 < CHUNKS1)
                def _():
                    pltpu.async_copy(
                        t_sh.at[src_v.at[ch + 2]], vals[b2ago], gs[b2ago])
            return carry

        lax.fori_loop(0, CHUNKS1 // 4, _scatter, 0)
        pltpu.make_async_copy(
            vals[2], acc_sh.at[dst_v.at[CHUNKS1 - 2]], ss[2]).wait()
        pltpu.make_async_copy(
            vals[3], acc_sh.at[dst_v.at[CHUNKS1 - 1]], ss[3]).wait()
        plsc.subcore_barrier()
        pltpu.sync_copy(acc_sh.at[pl.ds(s * STRIPE, STRIPE)], av)
        pltpu.sync_copy(dis_hbm.at[pl.ds(s * STRIPE, STRIPE)], dv)
        b2r = b2_v[...]

        def _combine(i, carry):
            a16 = av[pl.ds(i * 16, 16)]
            d16 = dv[pl.ds(i * 16, 16)]
            t16 = tv[pl.ds(i * 16, 16)]
            ov[pl.ds(i * 16, 16)] = d16 * (a16 + t16) + b2r
            return carry

        lax.fori_loop(0, STRIPE // 16, _combine, 0)
        pltpu.sync_copy(ov, out_hbm.at[pl.ds(s * STRIPE, STRIPE)])


def _tc1_body(x_ref, w_ref, dg_ref, y_ref, dis_ref):
    deg = 1.0 + dg_ref[:, 0:1] + dg_ref[:, 1:2]
    dis = lax.rsqrt(deg)
    xw = jnp.dot(x_ref[...], w_ref[...], preferred_element_type=jnp.float32)
    y_ref[...] = xw * dis
    dis_ref[...] = dis


def _tc1(xp, W1, degp_t):
    blk = 1024
    grid = NP // blk
    return pl.pallas_call(
        _tc1_body,
        grid=(grid,),
        in_specs=[
            pl.BlockSpec((blk, 128), lambda i: (i, 0)),
            pl.BlockSpec((128, 128), lambda i: (0, 0)),
            pl.BlockSpec((blk, 2), lambda i: (i, 0)),
        ],
        out_specs=[
            pl.BlockSpec((blk, 128), lambda i: (i, 0)),
            pl.BlockSpec((blk, 1), lambda i: (i, 0)),
        ],
        out_shape=[
            jax.ShapeDtypeStruct((NP, 128), jnp.float32),
            jax.ShapeDtypeStruct((NP, 1), jnp.float32),
        ],
    )(xp, W1, degp_t)


def _tc2_body(ap_ref, y_ref, dis_ref, b1_ref, w2_ref, t_ref):
    agg = ap_ref[0] + ap_ref[1] + y_ref[...]
    h = jnp.maximum(agg * dis_ref[...] + b1_ref[...], 0.0)
    srow = jnp.sum(h * w2_ref[...], axis=1, keepdims=True)
    t_ref[...] = srow * dis_ref[...]


def _tc2(aggp, y, dis, b1r, w2r):
    blk = 1024
    grid = NP // blk
    return pl.pallas_call(
        _tc2_body,
        grid=(grid,),
        in_specs=[
            pl.BlockSpec((2, blk, 128), lambda i: (0, i, 0)),
            pl.BlockSpec((blk, 128), lambda i: (i, 0)),
            pl.BlockSpec((blk, 1), lambda i: (i, 0)),
            pl.BlockSpec((1, 128), lambda i: (0, 0)),
            pl.BlockSpec((1, 128), lambda i: (0, 0)),
        ],
        out_specs=pl.BlockSpec((blk, 1), lambda i: (i, 0)),
        out_shape=jax.ShapeDtypeStruct((NP, 1), jnp.float32),
    )(aggp, y, dis, b1r, w2r)


def _pack2x16(idx2d):
    w = idx2d[:, :64] | (idx2d[:, 64:] << 16)
    return w.reshape(EROWS // 2, 128)


@jax.jit
def kernel(x, edge_index, W1, b1, W2, b2):
    pad = (jnp.arange(EP - E, dtype=jnp.int32) % (NP - N)) + N
    srcp = jnp.concatenate([edge_index[0], pad]).reshape(EROWS, 128)
    dstp = jnp.concatenate([edge_index[1], pad]).reshape(EROWS, 128)
    srcw = _pack2x16(srcp)
    dstw = _pack2x16(dstp)
    xp = jnp.pad(x, ((0, NP - N), (0, 0)))

    degp = _sc_degree(dstp)
    y, dis = _tc1(xp, W1, jnp.transpose(degp))
    aggp = _sc_row_agg(srcw, dstw, y)
    t = _tc2(aggp, y, dis, b1.reshape(1, 128), W2.reshape(1, 128))
    out_full = _sc_scalar_agg(
        srcp, dstp, t.reshape(-1), dis.reshape(-1),
        jnp.broadcast_to(b2, (16,)))
    return out_full[:N]
